# baseline (speedup 1.0000x reference)
import functools

import jax
import jax.numpy as jnp
from jax import lax
from jax.experimental import pallas as pl
from jax.experimental.pallas import tpu as pltpu
from jax.experimental.pallas import tpu_sc as plsc

_B, _D, _C = 4096, 100, 128
_R = 2
_NBUF = 4
_OFFS = (0, 16, 32, 48, 64, 80, 84)


def _make_sc_kernel():
    info = plsc.get_sparse_core_info()
    nc, ns = info.num_cores, info.num_subcores
    nw = nc * ns
    rows_w = _B // nw
    chunks = rows_w // _R
    mesh = plsc.VectorSubcoreMesh(core_axis_name="c", subcore_axis_name="s")

    @functools.partial(
        pl.kernel,
        mesh=mesh,
        out_type=jax.ShapeDtypeStruct((_B, _D * _C), jnp.float32),
        scratch_types=[
            pltpu.VMEM((rows_w, _D), jnp.int32),
            *[pltpu.VMEM((_R, _D * _C), jnp.float32) for _ in range(_NBUF)],
            *[pltpu.SemaphoreType.DMA for _ in range(_NBUF)],
        ],
        compiler_params=pltpu.CompilerParams(needs_layout_passes=False),
    )
    def k(x_hbm, out_hbm, idx_v, *bufsems):
        bufs = bufsems[:_NBUF]
        sems = bufsems[_NBUF:]
        wid = lax.axis_index("s") * nc + lax.axis_index("c")
        zeros16 = jnp.zeros((16,), jnp.float32)
        ones16 = jnp.ones((16,), jnp.float32)
        lane = lax.broadcasted_iota(jnp.int32, (16,), 0) * _C

        pltpu.sync_copy(x_hbm.at[pl.ds(wid * rows_w, rows_w)], idx_v)

        def zbody(i, carry):
            base = i * 64
            for buf in bufs:
                for r in range(_R):
                    for u in range(4):
                        buf[r, pl.ds(base + u * 16, 16)] = zeros16
            return carry

        lax.fori_loop(0, _D * _C // 64, zbody, 0)

        def scatter(buf, chunk, val16):
            for r in range(_R):
                rowv = jnp.full((16,), r, jnp.int32)
                for off in _OFFS:
                    vals = idx_v[chunk * _R + r, pl.ds(off, 16)]
                    cols = vals + lane + off * _C
                    plsc.store_scatter(buf, [rowv, cols], val16)

        def out_slice(i):
            return out_hbm.at[pl.ds(wid * rows_w + i * _R, _R)]

        def body(i, carry):
            for parity in range(_NBUF):
                buf, sem = bufs[parity], sems[parity]

                @pl.when(lax.rem(i, _NBUF) == parity)
                def _():
                    @pl.when(i >= _NBUF)
                    def _():
                        pltpu.make_async_copy(buf, out_slice(i - _NBUF), sem).wait()
                        scatter(buf, i - _NBUF, zeros16)

                    scatter(buf, i, ones16)
                    pltpu.async_copy(buf, out_slice(i), sem)

            return carry

        lax.fori_loop(0, chunks, body, 0)
        for parity in range(_NBUF):
            i = chunks - _NBUF + parity
            pltpu.make_async_copy(bufs[parity], out_slice(i), sems[parity]).wait()

    return k


def kernel(inputs):
    return _make_sc_kernel()(inputs.astype(jnp.int32))


# speedup vs baseline: 1.0307x; 1.0307x over previous
import functools

import jax
import jax.numpy as jnp
from jax import lax
from jax.experimental import pallas as pl
from jax.experimental.pallas import tpu as pltpu
from jax.experimental.pallas import tpu_sc as plsc

_B, _D, _C = 4096, 100, 128
_R = 2
_NBUF = 4
_OFFS = (0, 16, 32, 48, 64, 80, 84)


def _make_sc_kernel():
    info = plsc.get_sparse_core_info()
    nc, ns = info.num_cores, info.num_subcores
    nw = nc * ns
    rows_w = _B // nw
    chunks = rows_w // _R
    mesh = plsc.VectorSubcoreMesh(core_axis_name="c", subcore_axis_name="s")

    @functools.partial(
        pl.kernel,
        mesh=mesh,
        out_type=jax.ShapeDtypeStruct((_B, _D * _C), jnp.float32),
        scratch_types=[
            pltpu.VMEM((rows_w, _D), jnp.int32),
            *[pltpu.VMEM((_R, _D * _C), jnp.float32) for _ in range(_NBUF)],
            *[pltpu.SemaphoreType.DMA for _ in range(_NBUF)],
        ],
        compiler_params=pltpu.CompilerParams(needs_layout_passes=False),
    )
    def k(x_hbm, out_hbm, idx_v, *bufsems):
        bufs = bufsems[:_NBUF]
        sems = bufsems[_NBUF:]
        wid = lax.axis_index("s") * nc + lax.axis_index("c")
        zeros16 = jnp.zeros((16,), jnp.float32)
        ones16 = jnp.ones((16,), jnp.float32)
        lane = lax.broadcasted_iota(jnp.int32, (16,), 0) * _C

        idx_cp = pltpu.async_copy(
            x_hbm.at[pl.ds(wid * rows_w, rows_w)], idx_v, sems[0]
        )

        def zero_buf(buf):
            def zbody(i, carry):
                base = i * 64
                for r in range(_R):
                    for u in range(4):
                        buf[r, pl.ds(base + u * 16, 16)] = zeros16
                return carry

            lax.fori_loop(0, _D * _C // 64, zbody, 0)

        def scatter(buf, chunk, val16):
            for r in range(_R):
                rowv = jnp.full((16,), r, jnp.int32)
                for off in _OFFS:
                    vals = idx_v[chunk * _R + r, pl.ds(off, 16)]
                    cols = vals + lane + off * _C
                    plsc.store_scatter(buf, [rowv, cols], val16)

        def out_slice(i):
            return out_hbm.at[pl.ds(wid * rows_w + i * _R, _R)]

        for p in range(_NBUF):
            zero_buf(bufs[p])
            if p == 0:
                idx_cp.wait()
            scatter(bufs[p], p, ones16)
            pltpu.async_copy(bufs[p], out_slice(p), sems[p])

        def body(i, carry):
            for parity in range(_NBUF):
                buf, sem = bufs[parity], sems[parity]

                @pl.when(lax.rem(i, _NBUF) == parity)
                def _():
                    pltpu.make_async_copy(buf, out_slice(i - _NBUF), sem).wait()
                    scatter(buf, i - _NBUF, zeros16)
                    scatter(buf, i, ones16)
                    pltpu.async_copy(buf, out_slice(i), sem)

            return carry

        lax.fori_loop(_NBUF, chunks, body, 0)
        for parity in range(_NBUF):
            i = chunks - _NBUF + parity
            pltpu.make_async_copy(bufs[parity], out_slice(i), sems[parity]).wait()

    return k


def kernel(inputs):
    return _make_sc_kernel()(inputs.astype(jnp.int32))
